# baseline (speedup 1.0000x reference)
import jax
import jax.numpy as jnp
from jax import lax
from jax.experimental import pallas as pl
from jax.experimental.pallas import tpu as pltpu
from jax.experimental.pallas import tpu_sc as plsc

_NUM_VOCAB = 100000
_EMBED_DIM = 64
_BATCH = 16384

_info = plsc.get_sparse_core_info()
_NC, _NS = _info.num_cores, _info.num_subcores
_NW = _NC * _NS
_BPW = _BATCH // _NW


_NCHUNK = 4
_CH = _BPW // _NCHUNK


def _sc_body(word_hbm, context_hbm, encode_hbm, decode_hbm,
             out_w_hbm, out_c_hbm,
             idx_w_v, idx_c_v, rows_w_v, rows_c_v, *sems):
    wid = lax.axis_index("s") * _NC + lax.axis_index("c")
    base = wid * _BPW
    pltpu.sync_copy(word_hbm.at[pl.ds(base, _BPW)], idx_w_v)
    pltpu.sync_copy(context_hbm.at[pl.ds(base, _BPW)], idx_c_v)
    gather_sems = sems[: 2 * _NCHUNK]
    wb_sem_w, wb_sem_c = sems[2 * _NCHUNK], sems[2 * _NCHUNK + 1]

    tables = (
        (encode_hbm, idx_w_v, rows_w_v, out_w_hbm, wb_sem_w),
        (decode_hbm, idx_c_v, rows_c_v, out_c_hbm, wb_sem_c),
    )
    copies = []
    for ci in range(_NCHUNK):
        for t, (tab, idx_v, rows_v, _, _) in enumerate(tables):
            sem = gather_sems[ci * 2 + t]
            c = pltpu.async_copy(
                tab.at[idx_v.at[pl.ds(ci * _CH, _CH)]],
                rows_v.at[pl.ds(ci * _CH, _CH)],
                sem,
            )
            copies.append(c)
    wb = []
    k = 0
    for ci in range(_NCHUNK):
        for t, (_, _, rows_v, out_hbm, wb_sem) in enumerate(tables):
            copies[k].wait()
            k += 1
            wb.append(pltpu.async_copy(
                rows_v.at[pl.ds(ci * _CH, _CH)],
                out_hbm.at[pl.ds(base + ci * _CH, _CH)],
                wb_sem,
            ))
    for c in wb:
        c.wait()


@jax.jit
def _skipgram(word, context, encode_W, decode_W):
    mesh = plsc.VectorSubcoreMesh(core_axis_name="c", subcore_axis_name="s")
    f = pl.kernel(
        _sc_body,
        mesh=mesh,
        out_type=(
            jax.ShapeDtypeStruct((_BATCH, _EMBED_DIM), jnp.float32),
            jax.ShapeDtypeStruct((_BATCH, _EMBED_DIM), jnp.float32),
        ),
        scratch_types=[
            pltpu.VMEM((_BPW,), jnp.int32),
            pltpu.VMEM((_BPW,), jnp.int32),
            pltpu.VMEM((_BPW, _EMBED_DIM), jnp.float32),
            pltpu.VMEM((_BPW, _EMBED_DIM), jnp.float32),
        ] + [pltpu.SemaphoreType.DMA] * (2 * _NCHUNK + 2),
        compiler_params=pltpu.CompilerParams(use_tc_tiling_on_sc=False),
    )
    return f(word, context, encode_W, decode_W)


def kernel(word, context, encode_W, decode_W):
    return _skipgram(word, context, encode_W, decode_W)


# speedup vs baseline: 1.0006x; 1.0006x over previous
import jax
import jax.numpy as jnp
from jax import lax
from jax.experimental import pallas as pl
from jax.experimental.pallas import tpu as pltpu
from jax.experimental.pallas import tpu_sc as plsc

_NUM_VOCAB = 100000
_EMBED_DIM = 64
_BATCH = 16384

_info = plsc.get_sparse_core_info()
_NC, _NS = _info.num_cores, _info.num_subcores
_NW = _NC * _NS
_BPW = _BATCH // _NW


_NCHUNK = 4
_CH = _BPW // _NCHUNK


def _sc_body(word_hbm, context_hbm, encode_hbm, decode_hbm,
             out_w_hbm, out_c_hbm,
             idx_w_v, idx_c_v, rows_w_v, rows_c_v, *sems):
    wid = lax.axis_index("s") * _NC + lax.axis_index("c")
    base = wid * _BPW
    pltpu.sync_copy(word_hbm.at[pl.ds(base, _BPW)], idx_w_v)
    pltpu.sync_copy(context_hbm.at[pl.ds(base, _BPW)], idx_c_v)
    gather_sems = sems[: 2 * _NCHUNK]
    wb_sem_w, wb_sem_c = sems[2 * _NCHUNK], sems[2 * _NCHUNK + 1]

    tables = (
        (encode_hbm, idx_w_v, rows_w_v, out_w_hbm, wb_sem_w),
        (decode_hbm, idx_c_v, rows_c_v, out_c_hbm, wb_sem_c),
    )
    copies = []
    for ci in range(_NCHUNK):
        for t, (tab, idx_v, rows_v, _, _) in enumerate(tables):
            sem = gather_sems[ci * 2 + t]
            c = pltpu.async_copy(
                tab.at[idx_v.at[pl.ds(ci * _CH, _CH)]],
                rows_v.at[pl.ds(ci * _CH, _CH)],
                sem,
            )
            copies.append(c)
    wb = []
    k = 0
    for ci in range(_NCHUNK):
        for t, (_, _, rows_v, out_hbm, wb_sem) in enumerate(tables):
            copies[k].wait()
            k += 1
            wb.append(pltpu.async_copy(
                rows_v.at[pl.ds(ci * _CH, _CH)],
                out_hbm.at[pl.ds(base + ci * _CH, _CH)],
                wb_sem,
            ))
    for c in wb:
        c.wait()


@jax.jit
def _skipgram(word, context, encode_W, decode_W):
    mesh = plsc.VectorSubcoreMesh(core_axis_name="c", subcore_axis_name="s")
    f = pl.kernel(
        _sc_body,
        mesh=mesh,
        out_type=(
            jax.ShapeDtypeStruct((_BATCH, _EMBED_DIM), jnp.float32),
            jax.ShapeDtypeStruct((_BATCH, _EMBED_DIM), jnp.float32),
        ),
        scratch_types=[
            pltpu.VMEM((_BPW,), jnp.int32),
            pltpu.VMEM((_BPW,), jnp.int32),
            pltpu.VMEM((_BPW, _EMBED_DIM), jnp.float32),
            pltpu.VMEM((_BPW, _EMBED_DIM), jnp.float32),
        ] + [pltpu.SemaphoreType.DMA] * (2 * _NCHUNK + 2),
        compiler_params=pltpu.CompilerParams(
            use_tc_tiling_on_sc=False,
            skip_device_barrier=True,
            disable_bounds_checks=True,
            disable_semaphore_checks=True,
        ),
    )
    return f(word, context, encode_W, decode_W)


def kernel(word, context, encode_W, decode_W):
    return _skipgram(word, context, encode_W, decode_W)


# speedup vs baseline: 1.0089x; 1.0083x over previous
import jax
import jax.numpy as jnp
from jax import lax
from jax.experimental import pallas as pl
from jax.experimental.pallas import tpu as pltpu
from jax.experimental.pallas import tpu_sc as plsc

_NUM_VOCAB = 100000
_EMBED_DIM = 64
_BATCH = 16384

_info = plsc.get_sparse_core_info()
_NC, _NS = _info.num_cores, _info.num_subcores
_NW = _NC * _NS
_BPW = _BATCH // _NW

_NCHUNK = 4
_CH = _BPW // _NCHUNK


def _gather_body(idx_hbm, table_hbm, out_hbm, idx_v, rows_v, *sems):
    wid = lax.axis_index("s") * _NC + lax.axis_index("c")
    base = wid * _BPW
    pltpu.sync_copy(idx_hbm.at[pl.ds(base, _BPW)], idx_v)
    gathers = []
    for ci in range(_NCHUNK):
        gathers.append(pltpu.async_copy(
            table_hbm.at[idx_v.at[pl.ds(ci * _CH, _CH)]],
            rows_v.at[pl.ds(ci * _CH, _CH)],
            sems[ci],
        ))
    wb = []
    for ci in range(_NCHUNK):
        gathers[ci].wait()
        wb.append(pltpu.async_copy(
            rows_v.at[pl.ds(ci * _CH, _CH)],
            out_hbm.at[pl.ds(base + ci * _CH, _CH)],
            sems[_NCHUNK + ci % 2],
        ))
    for c in wb:
        c.wait()


def _gather_one(idx, table):
    mesh = plsc.VectorSubcoreMesh(core_axis_name="c", subcore_axis_name="s")
    f = pl.kernel(
        _gather_body,
        mesh=mesh,
        out_type=jax.ShapeDtypeStruct((_BATCH, _EMBED_DIM), jnp.float32),
        scratch_types=[
            pltpu.VMEM((_BPW,), jnp.int32),
            pltpu.VMEM((_BPW, _EMBED_DIM), jnp.float32),
        ] + [pltpu.SemaphoreType.DMA] * (_NCHUNK + 2),
        compiler_params=pltpu.CompilerParams(use_tc_tiling_on_sc=False),
    )
    return f(idx, table)


@jax.jit
def _skipgram(word, context, encode_W, decode_W):
    return (_gather_one(word, encode_W), _gather_one(context, decode_W))


def kernel(word, context, encode_W, decode_W):
    return _skipgram(word, context, encode_W, decode_W)


# speedup vs baseline: 1.8366x; 1.8205x over previous
import jax
import jax.numpy as jnp
from jax import lax
from jax.experimental import pallas as pl
from jax.experimental.pallas import tpu as pltpu
from jax.experimental.pallas import tpu_sc as plsc

_NUM_VOCAB = 100000
_EMBED_DIM = 64
_BATCH = 16384

_info = plsc.get_sparse_core_info()
_NC, _NS, _NL = _info.num_cores, _info.num_subcores, _info.num_lanes
_DPT = _EMBED_DIM // _NS
_OCH = 2048
_NOC = _BATCH // _OCH


def _do_table(idx_hbm, table_t_hbm, out_t_hbm, s,
              idx_v, row_v, ostage_v, isem, rsem, osems):
    pltpu.sync_copy(idx_hbm, idx_v)
    for j in range(_DPT):
        d = s * _DPT + j
        pltpu.async_copy(table_t_hbm.at[pl.ds(d, 1), :],
                         row_v, rsem).wait()
        writes = [None, None]
        for ci in range(_NOC):
            p = ci % 2
            if writes[p] is not None:
                writes[p].wait()
            def gather_group(g, carry):
                b = ci * _OCH + g * _NL
                y = plsc.load_gather(row_v, [jnp.zeros((_NL,), jnp.int32),
                                             idx_v[pl.ds(b, _NL)]])
                ostage_v[p, pl.ds(g * _NL, _NL)] = y
                return carry

            lax.fori_loop(0, _OCH // _NL, gather_group, 0, unroll=4)
            writes[p] = pltpu.async_copy(
                ostage_v.at[pl.ds(p, 1), :],
                out_t_hbm.at[pl.ds(d, 1), pl.ds(ci * _OCH, _OCH)],
                osems[p])
        for w in writes:
            if w is not None:
                w.wait()


def _sc_body(word_hbm, context_hbm, enc_t_hbm, dec_t_hbm,
             out_w_t_hbm, out_c_t_hbm,
             idx_v, row_v, ostage_v, isem, rsem, *osems):
    c = lax.axis_index("c")
    s = lax.axis_index("s")

    @pl.when(c == 0)
    def _():
        _do_table(word_hbm, enc_t_hbm, out_w_t_hbm, s,
                  idx_v, row_v, ostage_v, isem, rsem, osems)

    @pl.when(c == 1)
    def _():
        _do_table(context_hbm, dec_t_hbm, out_c_t_hbm, s,
                  idx_v, row_v, ostage_v, isem, rsem, osems)


@jax.jit
def _skipgram(word, context, encode_W, decode_W):
    mesh = plsc.VectorSubcoreMesh(core_axis_name="c", subcore_axis_name="s")
    f = pl.kernel(
        _sc_body,
        mesh=mesh,
        out_type=(
            jax.ShapeDtypeStruct((_EMBED_DIM, _BATCH), jnp.float32),
            jax.ShapeDtypeStruct((_EMBED_DIM, _BATCH), jnp.float32),
        ),
        scratch_types=[
            pltpu.VMEM((_BATCH,), jnp.int32),
            pltpu.VMEM((1, _NUM_VOCAB), jnp.float32),
            pltpu.VMEM((2, _OCH), jnp.float32),
            pltpu.SemaphoreType.DMA,
            pltpu.SemaphoreType.DMA,
            pltpu.SemaphoreType.DMA,
            pltpu.SemaphoreType.DMA,
        ],
        compiler_params=pltpu.CompilerParams(needs_layout_passes=False),
    )
    out_w_t, out_c_t = f(word, context, encode_W.T, decode_W.T)
    return (out_w_t.T, out_c_t.T)


def kernel(word, context, encode_W, decode_W):
    return _skipgram(word, context, encode_W, decode_W)


# speedup vs baseline: 1.9440x; 1.0585x over previous
import jax
import jax.numpy as jnp
from jax import lax
from jax.experimental import pallas as pl
from jax.experimental.pallas import tpu as pltpu
from jax.experimental.pallas import tpu_sc as plsc

_NUM_VOCAB = 100000
_EMBED_DIM = 64
_BATCH = 16384

_info = plsc.get_sparse_core_info()
_NC, _NS, _NL = _info.num_cores, _info.num_subcores, _info.num_lanes
_DPT = _EMBED_DIM // _NS
_OCH = 4096
_NOC = _BATCH // _OCH


def _do_table(idx_hbm, table_t_hbm, out_t_hbm, s,
              idx_v, row_v, ostage_v, isem, rsem, osems):
    pltpu.sync_copy(idx_hbm, idx_v)
    zero_v = jnp.zeros((_NL,), jnp.int32)
    for j in range(_DPT):
        d = s * _DPT + j
        pltpu.async_copy(table_t_hbm.at[pl.ds(d, 1), :],
                         row_v, rsem).wait()
        writes = [None, None]
        for ci in range(_NOC):
            p = ci % 2
            if writes[p] is not None:
                writes[p].wait()
            def gather_group(g, carry):
                b = ci * _OCH + g * _NL
                y = plsc.load_gather(row_v, [zero_v, idx_v[pl.ds(b, _NL)]])
                ostage_v[p, pl.ds(g * _NL, _NL)] = y
                return carry

            lax.fori_loop(0, _OCH // _NL, gather_group, 0, unroll=8)
            writes[p] = pltpu.async_copy(
                ostage_v.at[pl.ds(p, 1), :],
                out_t_hbm.at[pl.ds(d, 1), pl.ds(ci * _OCH, _OCH)],
                osems[p])
        for w in writes:
            if w is not None:
                w.wait()


def _sc_body(word_hbm, context_hbm, enc_t_hbm, dec_t_hbm,
             out_w_t_hbm, out_c_t_hbm,
             idx_v, row_v, ostage_v, isem, rsem, *osems):
    c = lax.axis_index("c")
    s = lax.axis_index("s")

    @pl.when(c == 0)
    def _():
        _do_table(word_hbm, enc_t_hbm, out_w_t_hbm, s,
                  idx_v, row_v, ostage_v, isem, rsem, osems)

    @pl.when(c == 1)
    def _():
        _do_table(context_hbm, dec_t_hbm, out_c_t_hbm, s,
                  idx_v, row_v, ostage_v, isem, rsem, osems)


@jax.jit
def _skipgram(word, context, encode_W, decode_W):
    mesh = plsc.VectorSubcoreMesh(core_axis_name="c", subcore_axis_name="s")
    f = pl.kernel(
        _sc_body,
        mesh=mesh,
        out_type=(
            jax.ShapeDtypeStruct((_EMBED_DIM, _BATCH), jnp.float32),
            jax.ShapeDtypeStruct((_EMBED_DIM, _BATCH), jnp.float32),
        ),
        scratch_types=[
            pltpu.VMEM((_BATCH,), jnp.int32),
            pltpu.VMEM((1, _NUM_VOCAB), jnp.float32),
            pltpu.VMEM((2, _OCH), jnp.float32),
            pltpu.SemaphoreType.DMA,
            pltpu.SemaphoreType.DMA,
            pltpu.SemaphoreType.DMA,
            pltpu.SemaphoreType.DMA,
        ],
        compiler_params=pltpu.CompilerParams(
            needs_layout_passes=False,
            skip_device_barrier=True,
            disable_bounds_checks=True,
            disable_semaphore_checks=True,
        ),
    )
    out_w_t, out_c_t = f(word, context, encode_W.T, decode_W.T)
    return (out_w_t.T, out_c_t.T)


def kernel(word, context, encode_W, decode_W):
    return _skipgram(word, context, encode_W, decode_W)


# speedup vs baseline: 1.9449x; 1.0005x over previous
import jax
import jax.numpy as jnp
from jax import lax
from jax.experimental import pallas as pl
from jax.experimental.pallas import tpu as pltpu
from jax.experimental.pallas import tpu_sc as plsc

_NUM_VOCAB = 100000
_EMBED_DIM = 64
_BATCH = 16384

_info = plsc.get_sparse_core_info()
_NC, _NS, _NL = _info.num_cores, _info.num_subcores, _info.num_lanes
_DPT = _EMBED_DIM // _NS
_OCH = 4096
_NOC = _BATCH // _OCH


def _do_table(idx_hbm, table_t_hbm, out_t_hbm, s,
              idx_v, row_v, ostage_v, isem, rsem, osems):
    pltpu.sync_copy(idx_hbm, idx_v)
    for j in range(_DPT):
        d = s * _DPT + j
        pltpu.async_copy(table_t_hbm.at[d, :],
                         row_v, rsem).wait()
        writes = [None, None]
        for ci in range(_NOC):
            p = ci % 2
            if writes[p] is not None:
                writes[p].wait()
            def gather_group(g, carry):
                b = ci * _OCH + g * _NL
                y = plsc.load_gather(row_v, [idx_v[pl.ds(b, _NL)]])
                ostage_v[p, pl.ds(g * _NL, _NL)] = y
                return carry

            lax.fori_loop(0, _OCH // _NL, gather_group, 0, unroll=8)
            writes[p] = pltpu.async_copy(
                ostage_v.at[pl.ds(p, 1), :],
                out_t_hbm.at[pl.ds(d, 1), pl.ds(ci * _OCH, _OCH)],
                osems[p])
        for w in writes:
            if w is not None:
                w.wait()


def _sc_body(word_hbm, context_hbm, enc_t_hbm, dec_t_hbm,
             out_w_t_hbm, out_c_t_hbm,
             idx_v, row_v, ostage_v, isem, rsem, *osems):
    c = lax.axis_index("c")
    s = lax.axis_index("s")

    @pl.when(c == 0)
    def _():
        _do_table(word_hbm, enc_t_hbm, out_w_t_hbm, s,
                  idx_v, row_v, ostage_v, isem, rsem, osems)

    @pl.when(c == 1)
    def _():
        _do_table(context_hbm, dec_t_hbm, out_c_t_hbm, s,
                  idx_v, row_v, ostage_v, isem, rsem, osems)


@jax.jit
def _skipgram(word, context, encode_W, decode_W):
    mesh = plsc.VectorSubcoreMesh(core_axis_name="c", subcore_axis_name="s")
    f = pl.kernel(
        _sc_body,
        mesh=mesh,
        out_type=(
            jax.ShapeDtypeStruct((_EMBED_DIM, _BATCH), jnp.float32),
            jax.ShapeDtypeStruct((_EMBED_DIM, _BATCH), jnp.float32),
        ),
        scratch_types=[
            pltpu.VMEM((_BATCH,), jnp.int32),
            pltpu.VMEM((_NUM_VOCAB,), jnp.float32),
            pltpu.VMEM((2, _OCH), jnp.float32),
            pltpu.SemaphoreType.DMA,
            pltpu.SemaphoreType.DMA,
            pltpu.SemaphoreType.DMA,
            pltpu.SemaphoreType.DMA,
        ],
        compiler_params=pltpu.CompilerParams(
            needs_layout_passes=False,
            skip_device_barrier=True,
            disable_bounds_checks=True,
            disable_semaphore_checks=True,
        ),
    )
    out_w_t, out_c_t = f(word, context, encode_W.T, decode_W.T)
    return (out_w_t.T, out_c_t.T)


def kernel(word, context, encode_W, decode_W):
    return _skipgram(word, context, encode_W, decode_W)
